# baseline (speedup 1.0000x reference)
import jax
import jax.numpy as jnp
from jax import lax
from jax.experimental import pallas as pl
from jax.experimental.pallas import tpu as pltpu
from jax.experimental.pallas import tpu_sc as plsc

B = 4096
INPUT_SIZE = 100000
DIM = 1024
NNZ = 204800

NC = 2
NS = 16
L = 16

NCALL = 4
QD = 128
K = 128
PER_SUB = NNZ // NS
NCHUNK = PER_SUB // K
ROWS_PER_SUB = B // NS


def _sc_body(w2_hbm, rows_hbm, cols_hbm, vals_hbm, out_hbm,
             acc_sp, idx_v, rows_v, vals_v, gat_v, gsem, ssem):
    c = lax.axis_index("c")
    s = lax.axis_index("s")

    pltpu.sync_copy(rows_hbm.at[s], rows_v)
    pltpu.sync_copy(cols_hbm.at[s], cols_v := idx_v)
    pltpu.sync_copy(vals_hbm.at[s], vals_v)

    def idx_init(j, cr):
        for m in range(K // L):
            cv = cols_v[j, pl.ds(m * L, L)]
            idx_v[j, pl.ds(m * L, L)] = cv * 2 + c
        return cr
    lax.fori_loop(0, NCHUNK, idx_init, 0)

    def gather_start(j, b):
        pltpu.async_copy(w2_hbm.at[idx_v.at[j]], gat_v.at[b], gsem.at[b])

    def gather_wait(j, b):
        pltpu.make_async_copy(w2_hbm.at[idx_v.at[j]], gat_v.at[b],
                              gsem.at[b]).wait()

    def scatter_start(j, b):
        pltpu.async_copy(gat_v.at[b], acc_sp.at[rows_v.at[j]], ssem.at[b],
                         add=True)

    def scatter_wait(j, b):
        pltpu.make_async_copy(gat_v.at[b], acc_sp.at[rows_v.at[j]],
                              ssem.at[b]).wait()

    def scale_chunk(j, b):
        def group(g, carry):
            vv = vals_v[j, pl.ds(g * L, L)]
            for l in range(L):
                r = g * L + l
                val = vv[l]
                for m in range(QD // L):
                    cur = gat_v[b, r, pl.ds(m * L, L)]
                    gat_v[b, r, pl.ds(m * L, L)] = cur * val
            return carry
        lax.fori_loop(0, K // L, group, 0)

    def zrow(r, cr):
        for m in range(QD // L):
            gat_v[0, r, pl.ds(m * L, L)] = jnp.zeros((L,), jnp.float32)
        return cr
    lax.fori_loop(0, K, zrow, 0)
    for t in range(ROWS_PER_SUB // K):
        pltpu.sync_copy(gat_v.at[0],
                        acc_sp.at[pl.ds(s * ROWS_PER_SUB + t * K, K)])
    plsc.subcore_barrier()

    gather_start(0, 0)

    def step(jj, cr):
        for b in range(2):
            j = jj * 2 + b
            gather_wait(j, b)

            @pl.when(j + 1 < NCHUNK)
            def _():
                @pl.when(j >= 1)
                def _():
                    scatter_wait(j - 1, 1 - b)
                gather_start(j + 1, 1 - b)

            scale_chunk(j, b)
            scatter_start(j, b)
        return cr
    lax.fori_loop(0, NCHUNK // 2, step, 0)

    scatter_wait(NCHUNK - 2, 0)
    scatter_wait(NCHUNK - 1, 1)
    plsc.subcore_barrier()

    pltpu.sync_copy(
        acc_sp.at[pl.ds(s * ROWS_PER_SUB, ROWS_PER_SUB)],
        out_hbm.at[c, pl.ds(s * ROWS_PER_SUB, ROWS_PER_SUB)])


def _sc_encode_slice(w2, rows3, cols3, vals3):
    mesh = plsc.VectorSubcoreMesh(core_axis_name="c", subcore_axis_name="s",
                                  num_cores=NC, num_subcores=NS)
    return pl.kernel(
        _sc_body,
        out_type=jax.ShapeDtypeStruct((NC, B, QD), jnp.float32),
        mesh=mesh,
        scratch_types=[
            pltpu.VMEM_SHARED((B, QD), jnp.float32),
            pltpu.VMEM((NCHUNK, K), jnp.int32),
            pltpu.VMEM((NCHUNK, K), jnp.int32),
            pltpu.VMEM((NCHUNK, K), jnp.float32),
            pltpu.VMEM((2, K, QD), jnp.float32),
            pltpu.SemaphoreType.DMA((2,)),
            pltpu.SemaphoreType.DMA((2,)),
        ],
    )(w2, rows3, cols3, vals3)


BS = 512


def _tc_body(x0_ref, x1_ref, x2_ref, x3_ref, w0_ref, w1_ref, w2_ref, o_ref):
    h = jnp.zeros((BS, DIM), jnp.float32)
    for j, x_ref in enumerate((x0_ref, x1_ref, x2_ref, x3_ref)):
        x = jnp.maximum(x_ref[...], 0.0)
        for cc in range(NC):
            h = h + jnp.dot(x[cc], w0_ref[2 * j + cc],
                            preferred_element_type=jnp.float32)
    h = jnp.maximum(h, 0.0)
    h = jnp.maximum(jnp.dot(h, w1_ref[...],
                            preferred_element_type=jnp.float32), 0.0)
    h = jnp.maximum(jnp.dot(h, w2_ref[...],
                            preferred_element_type=jnp.float32), 0.0)
    o_ref[...] = h


def _tc_mlp(encs, w0r, w1, w2):
    x_spec = pl.BlockSpec((NC, BS, QD), lambda i: (0, i, 0))
    return pl.pallas_call(
        _tc_body,
        grid=(B // BS,),
        in_specs=[
            x_spec, x_spec, x_spec, x_spec,
            pl.BlockSpec((2 * NCALL, QD, DIM), lambda i: (0, 0, 0)),
            pl.BlockSpec((DIM, DIM), lambda i: (0, 0)),
            pl.BlockSpec((DIM, DIM), lambda i: (0, 0)),
        ],
        out_specs=pl.BlockSpec((BS, DIM), lambda i: (i, 0)),
        out_shape=jax.ShapeDtypeStruct((B, DIM), jnp.float32),
    )(*encs, w0r, w1, w2)


@jax.jit
def kernel(coherence_indices, coherence_values, trans_weights,
           hidden_0, hidden_1, hidden_2):
    rows3 = coherence_indices[:, 0].astype(jnp.int32).reshape(NS, NCHUNK, K)
    cols3 = coherence_indices[:, 1].astype(jnp.int32).reshape(NS, NCHUNK, K)
    vals3 = coherence_values.reshape(NS, NCHUNK, K)
    encs = []
    for j in range(NCALL):
        wj = lax.slice_in_dim(trans_weights, 2 * QD * j, 2 * QD * (j + 1),
                              axis=1).reshape(NC * INPUT_SIZE, QD)
        encs.append(_sc_encode_slice(wj, rows3, cols3, vals3))
    w0r = hidden_0.reshape(2 * NCALL, QD, DIM)
    return _tc_mlp(encs, w0r, hidden_1, hidden_2)


# speedup vs baseline: 1.6358x; 1.6358x over previous
import jax
import jax.numpy as jnp
from jax import lax
from jax.experimental import pallas as pl
from jax.experimental.pallas import tpu as pltpu
from jax.experimental.pallas import tpu_sc as plsc

B = 4096
INPUT_SIZE = 100000
DIM = 1024
NNZ = 204800

NC = 2
NS = 16
L = 16

NQ = 8
QD = DIM // NQ
K = 128
PER_SUB = NNZ // NS
NCHUNK = PER_SUB // K
ROWS_PER_SUB = B // NS


def _sc_body(w4_hbm, rows_hbm, cols_hbm, vals_hbm, out_hbm,
             acc_sp, idx_v, rows_v, vals_v, gat_v, gsem, ssem):
    c = lax.axis_index("c")
    s = lax.axis_index("s")

    pltpu.sync_copy(rows_hbm.at[s], rows_v)
    pltpu.sync_copy(cols_hbm.at[s], cols_v := idx_v)
    pltpu.sync_copy(vals_hbm.at[s], vals_v)

    def idx_init(j, cr):
        for m in range(K // L):
            cv = cols_v[j, pl.ds(m * L, L)]
            idx_v[j, pl.ds(m * L, L)] = (
                ((cv >> 3) << 6) | (cv & 7)) + c * (NQ // 2) * NQ
        return cr
    lax.fori_loop(0, NCHUNK, idx_init, 0)

    def gather_start(j, b):
        pltpu.async_copy(w4_hbm.at[idx_v.at[j]], gat_v.at[b], gsem.at[b])

    def gather_wait(j, b):
        pltpu.make_async_copy(w4_hbm.at[idx_v.at[j]], gat_v.at[b],
                              gsem.at[b]).wait()

    def scatter_start(j, b):
        pltpu.async_copy(gat_v.at[b], acc_sp.at[rows_v.at[j]], ssem.at[b],
                         add=True)

    def scatter_wait(j, b):
        pltpu.make_async_copy(gat_v.at[b], acc_sp.at[rows_v.at[j]],
                              ssem.at[b]).wait()

    def scale_chunk(j, b):
        def group(g, carry):
            vv = vals_v[j, pl.ds(g * L, L)]
            for l in range(L):
                r = g * L + l
                val = vv[l]
                for m in range(QD // L):
                    cur = gat_v[b, r, pl.ds(m * L, L)]
                    gat_v[b, r, pl.ds(m * L, L)] = cur * val
            return carry
        lax.fori_loop(0, K // L, group, 0)

    def pass_body(p, carry):
        q = c * (NQ // 2) + p

        @pl.when(p >= 1)
        def _():
            def idx_bump(j, cr):
                for m in range(K // L):
                    cur = idx_v[j, pl.ds(m * L, L)]
                    idx_v[j, pl.ds(m * L, L)] = cur + NQ
                return cr
            lax.fori_loop(0, NCHUNK, idx_bump, 0)

        def zrow(r, cr):
            for m in range(QD // L):
                gat_v[0, r, pl.ds(m * L, L)] = jnp.zeros((L,), jnp.float32)
            return cr
        lax.fori_loop(0, K, zrow, 0)
        for t in range(ROWS_PER_SUB // K):
            pltpu.sync_copy(gat_v.at[0],
                            acc_sp.at[pl.ds(s * ROWS_PER_SUB + t * K, K)])
        plsc.subcore_barrier()

        gather_start(0, 0)

        def step(jj, cr):
            for b in range(2):
                j = jj * 2 + b
                gather_wait(j, b)

                @pl.when(j >= 1)
                def _():
                    scatter_wait(j - 1, 1 - b)

                @pl.when(j + 1 < NCHUNK)
                def _():
                    gather_start(j + 1, 1 - b)

                scale_chunk(j, b)
                scatter_start(j, b)
            return cr
        lax.fori_loop(0, NCHUNK // 2, step, 0)

        scatter_wait(NCHUNK - 1, 1)
        plsc.subcore_barrier()

        pltpu.sync_copy(
            acc_sp.at[pl.ds(s * ROWS_PER_SUB, ROWS_PER_SUB)],
            out_hbm.at[q, pl.ds(s * ROWS_PER_SUB, ROWS_PER_SUB)])
        return carry

    lax.fori_loop(0, NQ // 2, pass_body, 0)


def _sc_encode(w4, rows3, cols3, vals3):
    mesh = plsc.VectorSubcoreMesh(core_axis_name="c", subcore_axis_name="s",
                                  num_cores=NC, num_subcores=NS)
    return pl.kernel(
        _sc_body,
        out_type=jax.ShapeDtypeStruct((NQ, B, QD), jnp.float32),
        mesh=mesh,
        scratch_types=[
            pltpu.VMEM_SHARED((B, QD), jnp.float32),
            pltpu.VMEM((NCHUNK, K), jnp.int32),
            pltpu.VMEM((NCHUNK, K), jnp.int32),
            pltpu.VMEM((NCHUNK, K), jnp.float32),
            pltpu.VMEM((2, K, QD), jnp.float32),
            pltpu.SemaphoreType.DMA((2,)),
            pltpu.SemaphoreType.DMA((2,)),
        ],
    )(w4, rows3, cols3, vals3)


BS = 512


def _tc_body(x4_ref, w0_ref, w1_ref, w2_ref, o_ref):
    x = jnp.maximum(x4_ref[...], 0.0)
    h = jnp.zeros((BS, DIM), jnp.float32)
    for qq in range(NQ):
        h = h + jnp.dot(x[qq], w0_ref[qq],
                        preferred_element_type=jnp.float32)
    h = jnp.maximum(h, 0.0)
    h = jnp.maximum(jnp.dot(h, w1_ref[...],
                            preferred_element_type=jnp.float32), 0.0)
    h = jnp.maximum(jnp.dot(h, w2_ref[...],
                            preferred_element_type=jnp.float32), 0.0)
    o_ref[...] = h


def _tc_mlp(enc4, w0r, w1, w2):
    return pl.pallas_call(
        _tc_body,
        grid=(B // BS,),
        in_specs=[
            pl.BlockSpec((NQ, BS, QD), lambda i: (0, i, 0)),
            pl.BlockSpec((NQ, QD, DIM), lambda i: (0, 0, 0)),
            pl.BlockSpec((DIM, DIM), lambda i: (0, 0)),
            pl.BlockSpec((DIM, DIM), lambda i: (0, 0)),
        ],
        out_specs=pl.BlockSpec((BS, DIM), lambda i: (i, 0)),
        out_shape=jax.ShapeDtypeStruct((B, DIM), jnp.float32),
    )(enc4, w0r, w1, w2)


@jax.jit
def kernel(coherence_indices, coherence_values, trans_weights,
           hidden_0, hidden_1, hidden_2):
    rows3 = coherence_indices[:, 0].astype(jnp.int32).reshape(NS, NCHUNK, K)
    cols3 = coherence_indices[:, 1].astype(jnp.int32).reshape(NS, NCHUNK, K)
    vals3 = coherence_values.reshape(NS, NCHUNK, K)
    w4 = (trans_weights.reshape(INPUT_SIZE // 8, 8, NQ, QD)
          .transpose(0, 2, 1, 3).reshape(NQ * INPUT_SIZE, QD))
    enc4 = _sc_encode(w4, rows3, cols3, vals3)
    w0r = hidden_0.reshape(NQ, QD, DIM)
    return _tc_mlp(enc4, w0r, hidden_1, hidden_2)


# speedup vs baseline: 1.8065x; 1.1043x over previous
import jax
import jax.numpy as jnp
from jax import lax
from jax.experimental import pallas as pl
from jax.experimental.pallas import tpu as pltpu
from jax.experimental.pallas import tpu_sc as plsc

B = 4096
INPUT_SIZE = 100000
DIM = 1024
NNZ = 204800

NC = 2
NS = 16
L = 16

NQ = 8
QD = DIM // NQ
K = 128
PER_SUB = NNZ // NS
NCHUNK = PER_SUB // K
ROWS_PER_SUB = B // NS
NSLOT = 3


def _sc_body(w4_hbm, rows_hbm, cols_hbm, vals_hbm, out_hbm,
             acc_sp, idx_v, rows_v, vals_v, gat_v, gsem, ssem):
    c = lax.axis_index("c")
    s = lax.axis_index("s")

    pltpu.sync_copy(rows_hbm.at[s], rows_v)
    pltpu.sync_copy(cols_hbm.at[s], cols_v := idx_v)
    pltpu.sync_copy(vals_hbm.at[s], vals_v)

    def idx_init(j, cr):
        for m in range(K // L):
            cv = cols_v[j, pl.ds(m * L, L)]
            idx_v[j, pl.ds(m * L, L)] = (
                ((cv >> 3) << 6) | (cv & 7)) + c * (NQ // 2) * NQ
        return cr
    lax.fori_loop(0, NCHUNK, idx_init, 0)

    def gather_start(j, b):
        pltpu.async_copy(w4_hbm.at[idx_v.at[j]], gat_v.at[b], gsem.at[b])

    def gather_wait(j, b):
        pltpu.make_async_copy(w4_hbm.at[idx_v.at[j]], gat_v.at[b],
                              gsem.at[b]).wait()

    def scatter_start(j, b):
        pltpu.async_copy(gat_v.at[b], acc_sp.at[rows_v.at[j]], ssem.at[b],
                         add=True)

    def scatter_wait(j, b):
        pltpu.make_async_copy(gat_v.at[b], acc_sp.at[rows_v.at[j]],
                              ssem.at[b]).wait()

    def scale_chunk(j, b):
        def group(g, carry):
            vv = vals_v[j, pl.ds(g * L, L)]
            for l in range(L):
                r = g * L + l
                val = vv[l]
                for m in range(QD // L):
                    cur = gat_v[b, r, pl.ds(m * L, L)]
                    gat_v[b, r, pl.ds(m * L, L)] = cur * val
            return carry
        lax.fori_loop(0, K // L, group, 0)

    def pass_body(p, carry):
        q = c * (NQ // 2) + p

        @pl.when(p >= 1)
        def _():
            def idx_bump(j, cr):
                for m in range(K // L):
                    cur = idx_v[j, pl.ds(m * L, L)]
                    idx_v[j, pl.ds(m * L, L)] = cur + NQ
                return cr
            lax.fori_loop(0, NCHUNK, idx_bump, 0)

        def zrow(r, cr):
            for m in range(QD // L):
                gat_v[0, r, pl.ds(m * L, L)] = jnp.zeros((L,), jnp.float32)
            return cr
        lax.fori_loop(0, K, zrow, 0)
        for t in range(ROWS_PER_SUB // K):
            pltpu.sync_copy(gat_v.at[0],
                            acc_sp.at[pl.ds(s * ROWS_PER_SUB + t * K, K)])
        plsc.subcore_barrier()

        for pb in range(NSLOT - 1):
            gather_start(pb, pb)

        def step(jj, cr):
            for b in range(NSLOT):
                j = jj * NSLOT + b
                gather_wait(j, b)

                @pl.when(j >= 1)
                def _():
                    scatter_wait(j - 1, (b + NSLOT - 1) % NSLOT)

                @pl.when(j + NSLOT - 1 < NCHUNK)
                def _():
                    gather_start(j + NSLOT - 1, (b + NSLOT - 1) % NSLOT)

                scale_chunk(j, b)
                scatter_start(j, b)
            return cr
        lax.fori_loop(0, (NCHUNK - 1) // NSLOT, step, 0)

        jt = NCHUNK - 1
        bt = jt % NSLOT
        gather_wait(jt, bt)
        scatter_wait(jt - 1, (bt + NSLOT - 1) % NSLOT)
        scale_chunk(jt, bt)
        scatter_start(jt, bt)
        scatter_wait(jt, bt)
        plsc.subcore_barrier()

        pltpu.sync_copy(
            acc_sp.at[pl.ds(s * ROWS_PER_SUB, ROWS_PER_SUB)],
            out_hbm.at[q, pl.ds(s * ROWS_PER_SUB, ROWS_PER_SUB)])
        return carry

    lax.fori_loop(0, NQ // 2, pass_body, 0)


def _sc_encode(w4, rows3, cols3, vals3):
    mesh = plsc.VectorSubcoreMesh(core_axis_name="c", subcore_axis_name="s",
                                  num_cores=NC, num_subcores=NS)
    return pl.kernel(
        _sc_body,
        out_type=jax.ShapeDtypeStruct((NQ, B, QD), jnp.float32),
        mesh=mesh,
        scratch_types=[
            pltpu.VMEM_SHARED((B, QD), jnp.float32),
            pltpu.VMEM((NCHUNK, K), jnp.int32),
            pltpu.VMEM((NCHUNK, K), jnp.int32),
            pltpu.VMEM((NCHUNK, K), jnp.float32),
            pltpu.VMEM((NSLOT, K, QD), jnp.float32),
            pltpu.SemaphoreType.DMA((NSLOT,)),
            pltpu.SemaphoreType.DMA((NSLOT,)),
        ],
    )(w4, rows3, cols3, vals3)


BS = 512


def _tc_body(x4_ref, w0_ref, w1_ref, w2_ref, o_ref):
    x = jnp.maximum(x4_ref[...], 0.0)
    h = jnp.zeros((BS, DIM), jnp.float32)
    for qq in range(NQ):
        h = h + jnp.dot(x[qq], w0_ref[qq],
                        preferred_element_type=jnp.float32)
    h = jnp.maximum(h, 0.0)
    h = jnp.maximum(jnp.dot(h, w1_ref[...],
                            preferred_element_type=jnp.float32), 0.0)
    h = jnp.maximum(jnp.dot(h, w2_ref[...],
                            preferred_element_type=jnp.float32), 0.0)
    o_ref[...] = h


def _tc_mlp(enc4, w0r, w1, w2):
    return pl.pallas_call(
        _tc_body,
        grid=(B // BS,),
        in_specs=[
            pl.BlockSpec((NQ, BS, QD), lambda i: (0, i, 0)),
            pl.BlockSpec((NQ, QD, DIM), lambda i: (0, 0, 0)),
            pl.BlockSpec((DIM, DIM), lambda i: (0, 0)),
            pl.BlockSpec((DIM, DIM), lambda i: (0, 0)),
        ],
        out_specs=pl.BlockSpec((BS, DIM), lambda i: (i, 0)),
        out_shape=jax.ShapeDtypeStruct((B, DIM), jnp.float32),
    )(enc4, w0r, w1, w2)


@jax.jit
def kernel(coherence_indices, coherence_values, trans_weights,
           hidden_0, hidden_1, hidden_2):
    rows3 = coherence_indices[:, 0].astype(jnp.int32).reshape(NS, NCHUNK, K)
    cols3 = coherence_indices[:, 1].astype(jnp.int32).reshape(NS, NCHUNK, K)
    vals3 = coherence_values.reshape(NS, NCHUNK, K)
    w4 = (trans_weights.reshape(INPUT_SIZE // 8, 8, NQ, QD)
          .transpose(0, 2, 1, 3).reshape(NQ * INPUT_SIZE, QD))
    enc4 = _sc_encode(w4, rows3, cols3, vals3)
    w0r = hidden_0.reshape(NQ, QD, DIM)
    return _tc_mlp(enc4, w0r, hidden_1, hidden_2)
